# Initial kernel scaffold; baseline (speedup 1.0000x reference)
import jax
import jax.numpy as jnp
from jax.experimental import pallas as pl


def kernel(x, PE, I, de, m, params, edge_index):
    raise NotImplementedError("write your pallas kernel here")


# speedup vs baseline: 10.5762x; 10.5762x over previous
import functools
import jax
import jax.numpy as jnp
import numpy as np
from jax.experimental import pallas as pl

N = 10000
E = 160000
D = 128
H = 8
DH = 16
DD = 16
L = 3
FF = 256

_INTERPRET = False


def _ln_rows(x, g, b):
    mu = jnp.mean(x, -1, keepdims=True)
    v = jnp.mean((x - mu) ** 2, -1, keepdims=True)
    return (x - mu) / jnp.sqrt(v + 1e-5) * g + b


def _pre_body(x_ref, pe_ref, ehw_ref, lw_ref, lb_ref, out_ref):
    out_ref[...] = (
        jnp.dot(x_ref[...], ehw_ref[...], preferred_element_type=jnp.float32)
        + jnp.dot(pe_ref[...], lw_ref[...], preferred_element_type=jnp.float32)
        + lb_ref[...]
    )


def _bias_body(de_ref, m_ref, wd_ref, wm_ref, cb_ref, out_ref):
    out_ref[...] = (
        jnp.dot(de_ref[...], wd_ref[...], preferred_element_type=jnp.float32)
        + jnp.dot(m_ref[...], wm_ref[...], preferred_element_type=jnp.float32)
        + cb_ref[...]
    )


def _qkv_body(h_ref, wkv_ref, bkv_ref, wq_ref, bq_ref, kv_ref, q_ref):
    h = h_ref[...]
    kv_ref[...] = jnp.dot(h, wkv_ref[...], preferred_element_type=jnp.float32) + bkv_ref[...]
    q_ref[...] = jnp.dot(h, wq_ref[...], preferred_element_type=jnp.float32) + bq_ref[...]


def _node_body(h_ref, wv2_ref, z2_ref, i_ref, sz_ref, piw_ref, pib_ref,
               ow_ref, ob_ref, l1g_ref, l1b_ref, f1w_ref, f1b_ref,
               f2w_ref, f2b_ref, l2g_ref, l2b_ref, out_ref):
    wv = wv2_ref[0] + wv2_ref[1]
    z = z2_ref[0] + z2_ref[1]
    zb = jnp.dot(z, sz_ref[...], preferred_element_type=jnp.float32)
    att = wv / (zb + 1e-6)
    h = h_ref[...]
    h2 = att + jnp.dot(i_ref[...], piw_ref[...], preferred_element_type=jnp.float32) + pib_ref[...]
    h2 = jnp.dot(h2, ow_ref[...], preferred_element_type=jnp.float32) + ob_ref[...]
    h2 = h + h2
    h2 = _ln_rows(h2, l1g_ref[...], l1b_ref[...])
    hf = jnp.maximum(jnp.dot(h2, f1w_ref[...], preferred_element_type=jnp.float32) + f1b_ref[...], 0.0)
    hf = jnp.dot(hf, f2w_ref[...], preferred_element_type=jnp.float32) + f2b_ref[...]
    out_ref[...] = _ln_rows(h2 + hf, l2g_ref[...], l2b_ref[...])


def _post_body(h_ref, m1w_ref, m1b_ref, m2w_ref, m2b_ref, out_ref):
    t = jnp.dot(h_ref[...], m1w_ref[...], preferred_element_type=jnp.float32) + m1b_ref[...]
    alpha = 1.6732632423543772
    scale = 1.0507009873554805
    t = scale * jnp.where(t > 0, t, alpha * (jnp.exp(jnp.minimum(t, 0.0)) - 1.0))
    out_ref[...] = jnp.dot(t, m2w_ref[...], preferred_element_type=jnp.float32) + m2b_ref[...]


def _rows_spec(block, ncols):
    return pl.BlockSpec((block, ncols), lambda i: (i, 0))


def _full_spec(shape):
    return pl.BlockSpec(shape, lambda i: tuple(0 for _ in shape))


def _call_rows(body, nrows, block, in_arrays, in_colspecs, out_shapes):
    grid = (nrows // block,)
    in_specs = []
    for a, c in zip(in_arrays, in_colspecs):
        if c == 'rows':
            in_specs.append(_rows_spec(block, a.shape[-1]))
        elif c == 'rows3':
            in_specs.append(pl.BlockSpec((a.shape[0], block, a.shape[2]),
                                         lambda i: (0, i, 0)))
        else:
            in_specs.append(_full_spec(a.shape))
    out_specs = jax.tree.map(lambda s: _rows_spec(block, s.shape[-1]), out_shapes)
    return pl.pallas_call(
        body,
        grid=grid,
        in_specs=in_specs,
        out_specs=out_specs,
        out_shape=out_shapes,
        interpret=_INTERPRET,
    )(*in_arrays)


def _edge_phase(KV, Q, bias_l, src, dst):
    Kh = KV[:, :D]
    Vh = KV[:, D:]
    dots = jnp.sum((Kh[src] * Q[dst]).reshape(E, H, DH), -1)
    s = jnp.exp(jnp.clip(4.0 * dots + bias_l, -10.0, 10.0)) * 0.5
    sV = (Vh[src].reshape(E, H, DH) * s[:, :, None]).reshape(E, D)
    wV = jax.ops.segment_sum(sV, dst, num_segments=N)
    z = jax.ops.segment_sum(s, dst, num_segments=N)
    wv2 = jnp.stack([wV, jnp.zeros_like(wV)])
    z16 = jnp.pad(z, ((0, 0), (0, 8)))
    z2 = jnp.stack([z16, jnp.zeros_like(z16)])
    return wv2, z2


def kernel(x, PE, I, de, m, params, edge_index):
    p = params
    src = edge_index[0]
    dst = edge_index[1]

    pdw_cs = p['pdw'].reshape(L, D, H, DH).sum(-1)
    pmw_cs = p['pmw'].reshape(L, D, H, DH).sum(-1)
    Wd = jnp.einsum('dk,lkh->dlh', p['emb_de_w'], pdw_cs).reshape(DD, L * H)
    Wm = jnp.einsum('dk,lkh->dlh', p['emb_m_w'], pmw_cs).reshape(8, L * H)
    cb = (jnp.einsum('k,lkh->lh', p['emb_de_b'], pdw_cs)
          + jnp.einsum('k,lkh->lh', p['emb_m_b'], pmw_cs)
          + p['pdb'].reshape(L, H, DH).sum(-1)
          + p['pmb'].reshape(L, H, DH).sum(-1)).reshape(1, L * H)
    Wd32 = jnp.pad(Wd, ((0, 0), (0, 32 - L * H)))
    Wm32 = jnp.pad(Wm, ((0, 0), (0, 32 - L * H)))
    cb32 = jnp.pad(cb, ((0, 0), (0, 32 - L * H)))
    Sz = (jnp.arange(128)[None, :] // DH == jnp.arange(16)[:, None]).astype(jnp.float32)

    h = _call_rows(
        _pre_body, N, 2000,
        [x, PE, p['emb_h_w'], p['lap_w'], p['lap_b'].reshape(1, D)],
        ['rows', 'rows', 'w', 'w', 'w'],
        jax.ShapeDtypeStruct((N, D), jnp.float32),
    )

    bias_all = _call_rows(
        _bias_body, E, 8000,
        [de, m, Wd32, Wm32, cb32],
        ['rows', 'rows', 'w', 'w', 'w'],
        jax.ShapeDtypeStruct((E, 32), jnp.float32),
    )
    bias_lhe = bias_all[:, :L * H].reshape(E, L, H).transpose(1, 0, 2)

    for l in range(L):
        Wkv = jnp.concatenate([p['Kw'][l], p['Vw'][l]], axis=1)
        bkv = jnp.concatenate([p['Kb'][l], p['Vb'][l]]).reshape(1, 2 * D)
        KV, Q = _call_rows(
            _qkv_body, N, 2000,
            [h, Wkv, bkv, p['Qw'][l], p['Qb'][l].reshape(1, D)],
            ['rows', 'w', 'w', 'w', 'w'],
            (jax.ShapeDtypeStruct((N, 2 * D), jnp.float32),
             jax.ShapeDtypeStruct((N, D), jnp.float32)),
        )

        wv2, z2 = _edge_phase(KV, Q, bias_lhe[l], src, dst)

        h = _call_rows(
            _node_body, N, 2000,
            [h, wv2, z2, I, Sz,
             p['piw'][l], p['pib'][l].reshape(1, D),
             p['Ow'][l], p['Ob'][l].reshape(1, D),
             p['ln1g'][l].reshape(1, D), p['ln1b'][l].reshape(1, D),
             p['f1w'][l], p['f1b'][l].reshape(1, FF),
             p['f2w'][l], p['f2b'][l].reshape(1, D),
             p['ln2g'][l].reshape(1, D), p['ln2b'][l].reshape(1, D)],
            ['rows', 'rows3', 'rows3', 'rows', 'w',
             'w', 'w', 'w', 'w', 'w', 'w', 'w', 'w', 'w', 'w', 'w', 'w'],
            jax.ShapeDtypeStruct((N, D), jnp.float32),
        )

    xh = _call_rows(
        _post_body, N, 2000,
        [h, p['m1w'], p['m1b'].reshape(1, 128), p['m2w'], p['m2b'].reshape(1, 128)],
        ['rows', 'w', 'w', 'w', 'w'],
        jax.ShapeDtypeStruct((N, 128), jnp.float32),
    )
    return (h, xh)


# speedup vs baseline: 11.1684x; 1.0560x over previous
import functools
import jax
import jax.numpy as jnp
import numpy as np
from jax import lax
from jax.experimental import pallas as pl
from jax.experimental.pallas import tpu as pltpu
from jax.experimental.pallas import tpu_sc as plsc

N = 10000
E = 160000
D = 128
H = 8
DH = 16
DD = 16
L = 3
FF = 256

NC = 2
NS = 16
NW = NC * NS
C = 128
CHUNKS_PER_W = 40
E_PAD = NW * CHUNKS_PER_W * C
N_PAD = 10240

_INTERPRET = False

HEAD_LANE4 = (0, 8, 4, 12)
LANE_HEAD4 = (0, 0, 0, 0, 2, 2, 2, 2, 1, 1, 1, 1, 3, 3, 3, 3)

_GDN = lax.GatherDimensionNumbers(
    offset_dims=(), collapsed_slice_dims=(0,), start_index_map=(0,))


def _shuf(x, s):
    perm = (lax.iota(jnp.int32, 16) ^ s).reshape(16, 1)
    return lax.gather(x, perm, _GDN, (1,),
                      mode=lax.GatherScatterMode.PROMISE_IN_BOUNDS)


def _packed_head_sums4(p, i16):
    r = [x + _shuf(x, 8) for x in p]
    m8 = i16 < 8
    c = [jnp.where(m8, r[0], _shuf(r[1], 8)),
         jnp.where(m8, r[2], _shuf(r[3], 8))]
    c = [x + _shuf(x, 4) for x in c]
    m4 = (i16 & 4) == 0
    d = jnp.where(m4, c[0], _shuf(c[1], 4))
    d = d + _shuf(d, 2)
    return d + _shuf(d, 1)


def _ln_rows(x, g, b):
    mu = jnp.mean(x, -1, keepdims=True)
    v = jnp.mean((x - mu) ** 2, -1, keepdims=True)
    return (x - mu) / jnp.sqrt(v + 1e-5) * g + b


def _pre_body(x_ref, pe_ref, ehw_ref, lw_ref, lb_ref, out_ref):
    out_ref[...] = (
        jnp.dot(x_ref[...], ehw_ref[...], preferred_element_type=jnp.float32)
        + jnp.dot(pe_ref[...], lw_ref[...], preferred_element_type=jnp.float32)
        + lb_ref[...]
    )


def _bias_body(de_ref, m_ref, wd_ref, wm_ref, cb_ref, out_ref):
    out_ref[...] = (
        jnp.dot(de_ref[...], wd_ref[...], preferred_element_type=jnp.float32)
        + jnp.dot(m_ref[...], wm_ref[...], preferred_element_type=jnp.float32)
        + cb_ref[...]
    )


def _qkv_body(h_ref, wkv_ref, bkv_ref, wq_ref, bq_ref, kv_ref, q_ref):
    h = h_ref[...]
    kv_ref[...] = jnp.dot(h, wkv_ref[...], preferred_element_type=jnp.float32) + bkv_ref[...]
    q_ref[...] = jnp.dot(h, wq_ref[...], preferred_element_type=jnp.float32) + bq_ref[...]


def _node_body(h_ref, att2_ref, i_ref, piw_ref, pib_ref,
               ow_ref, ob_ref, l1g_ref, l1b_ref, f1w_ref, f1b_ref,
               f2w_ref, f2b_ref, l2g_ref, l2b_ref, out_ref):
    att = jnp.concatenate([att2_ref[0], att2_ref[1]], axis=1)
    h = h_ref[...]
    h2 = att + jnp.dot(i_ref[...], piw_ref[...], preferred_element_type=jnp.float32) + pib_ref[...]
    h2 = jnp.dot(h2, ow_ref[...], preferred_element_type=jnp.float32) + ob_ref[...]
    h2 = h + h2
    h2 = _ln_rows(h2, l1g_ref[...], l1b_ref[...])
    hf = jnp.maximum(jnp.dot(h2, f1w_ref[...], preferred_element_type=jnp.float32) + f1b_ref[...], 0.0)
    hf = jnp.dot(hf, f2w_ref[...], preferred_element_type=jnp.float32) + f2b_ref[...]
    out_ref[...] = _ln_rows(h2 + hf, l2g_ref[...], l2b_ref[...])


def _post_body(h_ref, m1w_ref, m1b_ref, m2w_ref, m2b_ref, out_ref):
    t = jnp.dot(h_ref[...], m1w_ref[...], preferred_element_type=jnp.float32) + m1b_ref[...]
    alpha = 1.6732632423543772
    scale = 1.0507009873554805
    t = scale * jnp.where(t > 0, t, alpha * (jnp.exp(jnp.minimum(t, 0.0)) - 1.0))
    out_ref[...] = jnp.dot(t, m2w_ref[...], preferred_element_type=jnp.float32) + m2b_ref[...]


def _rows_spec(block, ncols):
    return pl.BlockSpec((block, ncols), lambda i: (i, 0))


def _full_spec(shape):
    return pl.BlockSpec(shape, lambda i: tuple(0 for _ in shape))


def _call_rows(body, nrows, block, in_arrays, in_colspecs, out_shapes):
    grid = (nrows // block,)
    in_specs = []
    for a, c in zip(in_arrays, in_colspecs):
        if c == 'rows':
            in_specs.append(_rows_spec(block, a.shape[-1]))
        elif c == 'rows3':
            in_specs.append(pl.BlockSpec((a.shape[0], block, a.shape[2]),
                                         lambda i: (0, i, 0)))
        else:
            in_specs.append(_full_spec(a.shape))
    out_specs = jax.tree.map(lambda s: _rows_spec(block, s.shape[-1]), out_shapes)
    return pl.pallas_call(
        body,
        grid=grid,
        in_specs=in_specs,
        out_specs=out_specs,
        out_shape=out_shapes,
        interpret=_INTERPRET,
    )(*in_arrays)


CHUNKS_PER_TILE = E_PAD // NS // C


def _edge_phase_jnp(KV, Q, bias_l, src, dst):
    Kh = KV[:, :D]
    Vh = KV[:, D:]
    dots = jnp.sum((Kh[src] * Q[dst]).reshape(E, H, DH), -1)
    sc = jnp.exp(jnp.clip(4.0 * dots + bias_l, -10.0, 10.0)) * 0.5
    sV = (Vh[src].reshape(E, H, DH) * sc[:, :, None]).reshape(E, D)
    wV = jax.ops.segment_sum(sV, dst, num_segments=N)
    z = jax.ops.segment_sum(sc, dst, num_segments=N)
    att = (wV.reshape(N, H, DH) / (z[:, :, None] + 1e-6)).reshape(N, D)
    att2 = jnp.stack([att[:, :64], att[:, 64:]])
    return jnp.pad(att2, ((0, 0), (0, N_PAD - N), (0, 0)))


def kernel(x, PE, I, de, m, params, edge_index):
    p = params
    src = edge_index[0]
    dst = edge_index[1]

    pdw_cs = p['pdw'].reshape(L, D, H, DH).sum(-1)
    pmw_cs = p['pmw'].reshape(L, D, H, DH).sum(-1)
    Wd = jnp.einsum('dk,lkh->dlh', p['emb_de_w'], pdw_cs).reshape(DD, L * H)
    Wm = jnp.einsum('dk,lkh->dlh', p['emb_m_w'], pmw_cs).reshape(8, L * H)
    cb = (jnp.einsum('k,lkh->lh', p['emb_de_b'], pdw_cs)
          + jnp.einsum('k,lkh->lh', p['emb_m_b'], pmw_cs)
          + p['pdb'].reshape(L, H, DH).sum(-1)
          + p['pmb'].reshape(L, H, DH).sum(-1)).reshape(1, L * H)
    Wd32 = jnp.pad(Wd, ((0, 0), (0, 32 - L * H)))
    Wm32 = jnp.pad(Wm, ((0, 0), (0, 32 - L * H)))
    cb32 = jnp.pad(cb, ((0, 0), (0, 32 - L * H)))

    h = _call_rows(
        _pre_body, N, 2000,
        [x, PE, p['emb_h_w'], p['lap_w'], p['lap_b'].reshape(1, D)],
        ['rows', 'rows', 'w', 'w', 'w'],
        jax.ShapeDtypeStruct((N, D), jnp.float32),
    )

    bias_all = _call_rows(
        _bias_body, E, 8000,
        [de, m, Wd32, Wm32, cb32],
        ['rows', 'rows', 'w', 'w', 'w'],
        jax.ShapeDtypeStruct((E, 32), jnp.float32),
    )
    bias_lhe = bias_all[:, :L * H].reshape(E, L, H).transpose(1, 0, 2)

    for l in range(L):
        Wkv = jnp.concatenate([p['Kw'][l], p['Vw'][l]], axis=1)
        bkv = jnp.concatenate([p['Kb'][l], p['Vb'][l]]).reshape(1, 2 * D)
        KV, Q = _call_rows(
            _qkv_body, N, 2000,
            [h, Wkv, bkv, p['Qw'][l], p['Qb'][l].reshape(1, D)],
            ['rows', 'w', 'w', 'w', 'w'],
            (jax.ShapeDtypeStruct((N, 2 * D), jnp.float32),
             jax.ShapeDtypeStruct((N, D), jnp.float32)),
        )

        att2 = _edge_phase_jnp(KV, Q, bias_lhe[l], src, dst)

        h = _call_rows(
            _node_body, N, 2000,
            [h, att2, I,
             p['piw'][l], p['pib'][l].reshape(1, D),
             p['Ow'][l], p['Ob'][l].reshape(1, D),
             p['ln1g'][l].reshape(1, D), p['ln1b'][l].reshape(1, D),
             p['f1w'][l], p['f1b'][l].reshape(1, FF),
             p['f2w'][l], p['f2b'][l].reshape(1, D),
             p['ln2g'][l].reshape(1, D), p['ln2b'][l].reshape(1, D)],
            ['rows', 'rows3', 'rows',
             'w', 'w', 'w', 'w', 'w', 'w', 'w', 'w', 'w', 'w', 'w', 'w'],
            jax.ShapeDtypeStruct((N, D), jnp.float32),
        )

    xh = _call_rows(
        _post_body, N, 2000,
        [h, p['m1w'], p['m1b'].reshape(1, 128), p['m2w'], p['m2b'].reshape(1, 128)],
        ['rows', 'w', 'w', 'w', 'w'],
        jax.ShapeDtypeStruct((N, 128), jnp.float32),
    )
    return (h, xh)
